# Initial kernel scaffold; baseline (speedup 1.0000x reference)
import jax
import jax.numpy as jnp
from jax.experimental import pallas as pl


def kernel(image_embeddings, scrna_embeddings, scrna_expressions, params):
    raise NotImplementedError("write your pallas kernel here")


# speedup vs baseline: 1.1556x; 1.1556x over previous
import functools

import jax
import jax.numpy as jnp
from jax import lax
from jax.experimental import pallas as pl
from jax.experimental.pallas import tpu as pltpu
from jax.experimental.pallas import tpu_sc as plsc

B = 256
D = 768
N = 20000
G = 2000
H = 512
K = 10
S = 11
NH = 8
DH = H // NH

TILE = 2048
NPAD = 20480
NT = NPAD // TILE


def _pc(body, **kw):
    return pl.pallas_call(body, **kw)


def _cls_body(x_ref, w1, b1, g1, be1, w2, b2, g2, be2, w3, b3, out_ref):
    def bn(h, g, b):
        mu = jnp.mean(h, axis=0, keepdims=True)
        var = jnp.mean((h - mu) ** 2, axis=0, keepdims=True)
        return g[...] * (h - mu) / jnp.sqrt(var + 1e-5) + b[...]

    h = jnp.maximum(jnp.dot(x_ref[...], w1[...], preferred_element_type=jnp.float32) + b1[...], 0.0)
    h = bn(h, g1, be1)
    h = jnp.maximum(jnp.dot(h, w2[...], preferred_element_type=jnp.float32) + b2[...], 0.0)
    h = bn(h, g2, be2)
    out_ref[...] = jnp.dot(h, w3[...], preferred_element_type=jnp.float32) + b3[...]


def _classifier(x, p):
    r2 = lambda a: a.reshape(1, -1)
    return _pc(
        _cls_body,
        out_shape=jax.ShapeDtypeStruct((B, 100), jnp.float32),
    )(x, p['cls_w1'], r2(p['cls_b1']), r2(p['cls_g1']), r2(p['cls_be1']),
      p['cls_w2'], r2(p['cls_b2']), r2(p['cls_g2']), r2(p['cls_be2']),
      p['cls_w3'], r2(p['cls_b3']))


def _retr_body(img_ref, db_ref, w_ref, idx_ref, sims_ref, qn_ref):
    i = pl.program_id(0)

    @pl.when(i == 0)
    def _():
        x = img_ref[...]
        nrm = jnp.sqrt(jnp.sum(x * x, axis=1, keepdims=True))
        qn_ref[...] = x / (nrm + 1e-8)

    d = db_ref[...]
    nrm = jnp.sqrt(jnp.sum(d * d, axis=1, keepdims=True))
    dn = d / (nrm + 1e-8)
    blk = lax.dot_general(qn_ref[...], dn, (((1,), (1,)), ((), ())),
                          preferred_element_type=jnp.float32)
    sims_ref[:, pl.ds(i * TILE, TILE)] = blk

    @pl.when(i == NT - 1)
    def _():
        RB = 32
        for rb in range(B // RB):
            s = sims_ref[rb * RB:(rb + 1) * RB, :]
            colid = lax.broadcasted_iota(jnp.int32, (RB, NPAD), 1)
            s = jnp.where(colid < N, s, -jnp.inf)
            vals, idxs = [], []
            for _k in range(K):
                m = jnp.max(s, axis=1, keepdims=True)
                ix = jnp.min(jnp.where(s == m, colid, jnp.int32(2 ** 30)),
                             axis=1, keepdims=True)
                vals.append(m)
                idxs.append(ix)
                s = jnp.where(colid == ix, -jnp.inf, s)
            v = jnp.concatenate(vals, axis=1)
            mm = jnp.max(v, axis=1, keepdims=True)
            e = jnp.exp(v - mm)
            w_ref[rb * RB:(rb + 1) * RB, :] = e / jnp.sum(e, axis=1, keepdims=True)
            idx_ref[rb * RB:(rb + 1) * RB, :] = jnp.concatenate(idxs, axis=1)


def _retrieval(img, db):
    return _pc(
        _retr_body,
        grid=(NT,),
        in_specs=[
            pl.BlockSpec((B, D), lambda i: (0, 0)),
            pl.BlockSpec((TILE, D), lambda i: (i, 0)),
        ],
        out_specs=[
            pl.BlockSpec((B, K), lambda i: (0, 0)),
            pl.BlockSpec((B, K), lambda i: (0, 0)),
        ],
        out_shape=[
            jax.ShapeDtypeStruct((B, K), jnp.float32),
            jax.ShapeDtypeStruct((B, K), jnp.int32),
        ],
        scratch_shapes=[
            pltpu.VMEM((B, NPAD), jnp.float32),
            pltpu.VMEM((B, D), jnp.float32),
        ],
    )(img, db)


NW = 32
BPW = (B * K) // NW
CH = 16


def _sc_gather(table, idx_flat):
    mesh = plsc.VectorSubcoreMesh(core_axis_name="c", subcore_axis_name="s")

    @functools.partial(
        pl.kernel,
        mesh=mesh,
        out_type=jax.ShapeDtypeStruct((B * K, G), jnp.float32),
        compiler_params=pltpu.CompilerParams(use_tc_tiling_on_sc=False),
        scratch_types=[
            pltpu.VMEM((CH,), jnp.int32),
            pltpu.VMEM((CH, G), jnp.float32),
            pltpu.SemaphoreType.DMA,
        ],
    )
    def k(table_hbm, idx_hbm, out_hbm, idx_v, rows_v, sem):
        wid = lax.axis_index("s") * 2 + lax.axis_index("c")
        base = wid * BPW
        for c in range(BPW // CH):
            pltpu.sync_copy(idx_hbm.at[pl.ds(base + c * CH, CH)], idx_v)
            pltpu.async_copy(table_hbm.at[idx_v], rows_v, sem).wait()
            pltpu.sync_copy(rows_v, out_hbm.at[pl.ds(base + c * CH, CH)])

    return k(table, idx_flat)


def _build_body(gath_ref, wflat_ref, wp, bp, img_ref, wi, bi, pos_ref, x0_ref):
    g = jnp.dot(gath_ref[...], wp[...], preferred_element_type=jnp.float32)
    g = g * wflat_ref[...]
    imgf = jnp.dot(img_ref[...], wi[...], preferred_element_type=jnp.float32) + bi[...]
    x0_ref[0:B, :] = imgf + pos_ref[0:1, :]
    for s in range(1, S):
        x0_ref[s * B:(s + 1) * B, :] = (g[(s - 1) * B:s * B, :] + bp[...]
                                        + pos_ref[s:s + 1, :])


def _build_tokens(gath, w_flat, img, p, pos):
    r2 = lambda a: a.reshape(1, -1)
    return _pc(
        _build_body,
        out_shape=jax.ShapeDtypeStruct((S * B, H), jnp.float32),
    )(gath, w_flat, p['scrna_proj_w'], r2(p['scrna_proj_b']),
      img, p['img_proj_w'], r2(p['img_proj_b']), pos)


def _ln(x, g, b):
    mu = jnp.mean(x, axis=1, keepdims=True)
    var = jnp.mean((x - mu) ** 2, axis=1, keepdims=True)
    return g[...] * (x - mu) / jnp.sqrt(var + 1e-5) + b[...]


def _attn_body(x_ref, wqkv, bqkv, wo, bo, g1, b1, out_ref):
    x = x_ref[...]
    qkv = jnp.dot(x, wqkv[...], preferred_element_type=jnp.float32) + bqkv[...]
    q = qkv[:, 0:H]
    kk = qkv[:, H:2 * H]
    v = qkv[:, 2 * H:3 * H]

    r = lax.broadcasted_iota(jnp.int32, (H, NH), 0)
    c = lax.broadcasted_iota(jnp.int32, (H, NH), 1)
    M = (r // DH == c).astype(jnp.float32)
    rt = lax.broadcasted_iota(jnp.int32, (NH, H), 0)
    ct = lax.broadcasted_iota(jnp.int32, (NH, H), 1)
    MT = (rt == ct // DH).astype(jnp.float32)

    scale = 1.0 / 8.0
    o_parts = []
    for i in range(S):
        qi = q[i * B:(i + 1) * B, :]
        sij = []
        for j in range(S):
            kj = kk[j * B:(j + 1) * B, :]
            sij.append(jnp.dot(qi * kj, M, preferred_element_type=jnp.float32) * scale)
        m = sij[0]
        for j in range(1, S):
            m = jnp.maximum(m, sij[j])
        es = [jnp.exp(sv - m) for sv in sij]
        z = es[0]
        for j in range(1, S):
            z = z + es[j]
        zi = 1.0 / z
        oi = None
        for j in range(S):
            ab = jnp.dot(es[j] * zi, MT, preferred_element_type=jnp.float32)
            t = ab * v[j * B:(j + 1) * B, :]
            oi = t if oi is None else oi + t
        o_parts.append(oi)
    o = jnp.concatenate(o_parts, axis=0)
    attn = jnp.dot(o, wo[...], preferred_element_type=jnp.float32) + bo[...]
    out_ref[...] = _ln(x + attn, g1, b1)


def _ffn_body(x_ref, w1, b1, w2, b2, g2, bb2, out_ref):
    x = x_ref[...]
    h = jnp.maximum(jnp.dot(x, w1[...], preferred_element_type=jnp.float32) + b1[...], 0.0)
    f = jnp.dot(h, w2[...], preferred_element_type=jnp.float32) + b2[...]
    out_ref[...] = _ln(x + f, g2, bb2)


def _encoder_layer(x, lp):
    r2 = lambda a: a.reshape(1, -1)
    x = _pc(
        _attn_body,
        out_shape=jax.ShapeDtypeStruct((S * B, H), jnp.float32),
    )(x, lp['wqkv'], r2(lp['bqkv']), lp['wo'], r2(lp['bo']),
      r2(lp['ln1_g']), r2(lp['ln1_b']))
    x = _pc(
        _ffn_body,
        out_shape=jax.ShapeDtypeStruct((S * B, H), jnp.float32),
    )(x, lp['ffn_w1'], r2(lp['ffn_b1']), lp['ffn_w2'], r2(lp['ffn_b2']),
      r2(lp['ln2_g']), r2(lp['ln2_b']))
    return x


def _head_body(x_ref, w1, b1, w2, b2, out_ref):
    o = jnp.maximum(jnp.dot(x_ref[...], w1[...], preferred_element_type=jnp.float32) + b1[...], 0.0)
    out_ref[...] = jnp.dot(o, w2[...], preferred_element_type=jnp.float32) + b2[...]


def _head(x, p):
    r2 = lambda a: a.reshape(1, -1)
    return _pc(
        _head_body,
        out_shape=jax.ShapeDtypeStruct((B, G), jnp.float32),
    )(x, p['out_w1'], r2(p['out_b1']), p['out_w2'], r2(p['out_b2']))


def kernel(image_embeddings, scrna_embeddings, scrna_expressions, params):
    p = params
    logits = _classifier(image_embeddings, p)

    weights, top_idx = _retrieval(image_embeddings, scrna_embeddings)
    idx_flat = top_idx.T.reshape(B * K)
    w_flat = weights.T.reshape(B * K, 1)

    gath = _sc_gather(scrna_expressions, idx_flat)

    pos = p['pos_emb'][0, :S, :]
    x = _build_tokens(gath, w_flat, image_embeddings, p, pos)
    for lp in p['layers']:
        x = _encoder_layer(x, lp)
    gene = _head(x[0:B], p)
    return gene, logits


# speedup vs baseline: 2.3011x; 1.9913x over previous
import functools

import jax
import jax.numpy as jnp
from jax import lax
from jax.experimental import pallas as pl
from jax.experimental.pallas import tpu as pltpu
from jax.experimental.pallas import tpu_sc as plsc

B = 256
D = 768
N = 20000
G = 2000
H = 512
K = 10
S = 11
NH = 8
DH = H // NH

TILE = 2048
NPAD = 20480
NT = NPAD // TILE


def _pc(body, **kw):
    return pl.pallas_call(body, **kw)


def _cls_body(x_ref, w1, b1, g1, be1, w2, b2, g2, be2, w3, b3, out_ref):
    def bn(h, g, b):
        mu = jnp.mean(h, axis=0, keepdims=True)
        var = jnp.mean((h - mu) ** 2, axis=0, keepdims=True)
        return g[...] * (h - mu) / jnp.sqrt(var + 1e-5) + b[...]

    h = jnp.maximum(jnp.dot(x_ref[...], w1[...], preferred_element_type=jnp.float32) + b1[...], 0.0)
    h = bn(h, g1, be1)
    h = jnp.maximum(jnp.dot(h, w2[...], preferred_element_type=jnp.float32) + b2[...], 0.0)
    h = bn(h, g2, be2)
    out_ref[...] = jnp.dot(h, w3[...], preferred_element_type=jnp.float32) + b3[...]


def _classifier(x, p):
    r2 = lambda a: a.reshape(1, -1)
    return _pc(
        _cls_body,
        out_shape=jax.ShapeDtypeStruct((B, 100), jnp.float32),
    )(x, p['cls_w1'], r2(p['cls_b1']), r2(p['cls_g1']), r2(p['cls_be1']),
      p['cls_w2'], r2(p['cls_b2']), r2(p['cls_g2']), r2(p['cls_be2']),
      p['cls_w3'], r2(p['cls_b3']))


def _retr_body(img_ref, db_ref, w_ref, idx_ref, sims_ref, qn_ref):
    i = pl.program_id(0)

    @pl.when(i == 0)
    def _():
        x = img_ref[...]
        nrm = jnp.sqrt(jnp.sum(x * x, axis=1, keepdims=True))
        qn_ref[...] = x / (nrm + 1e-8)

    d = db_ref[...]
    nrm = jnp.sqrt(jnp.sum(d * d, axis=1, keepdims=True))
    dn = d / (nrm + 1e-8)
    blk = lax.dot_general(qn_ref[...], dn, (((1,), (1,)), ((), ())),
                          preferred_element_type=jnp.float32)
    sims_ref[:, pl.ds(i * TILE, TILE)] = blk

    @pl.when(i == NT - 1)
    def _():
        RB = 32
        for rb in range(B // RB):
            s = sims_ref[rb * RB:(rb + 1) * RB, :]
            colid = lax.broadcasted_iota(jnp.int32, (RB, NPAD), 1)
            s = jnp.where(colid < N, s, -jnp.inf)
            vals, idxs = [], []
            for _k in range(K):
                m = jnp.max(s, axis=1, keepdims=True)
                ix = jnp.min(jnp.where(s == m, colid, jnp.int32(2 ** 30)),
                             axis=1, keepdims=True)
                vals.append(m)
                idxs.append(ix)
                s = jnp.where(colid == ix, -jnp.inf, s)
            v = jnp.concatenate(vals, axis=1)
            mm = jnp.max(v, axis=1, keepdims=True)
            e = jnp.exp(v - mm)
            w_ref[rb * RB:(rb + 1) * RB, :] = e / jnp.sum(e, axis=1, keepdims=True)
            idx_ref[rb * RB:(rb + 1) * RB, :] = jnp.concatenate(idxs, axis=1)


def _retrieval(img, db):
    return _pc(
        _retr_body,
        grid=(NT,),
        in_specs=[
            pl.BlockSpec((B, D), lambda i: (0, 0)),
            pl.BlockSpec((TILE, D), lambda i: (i, 0)),
        ],
        out_specs=[
            pl.BlockSpec((B, K), lambda i: (0, 0)),
            pl.BlockSpec((B, K), lambda i: (0, 0)),
        ],
        out_shape=[
            jax.ShapeDtypeStruct((B, K), jnp.float32),
            jax.ShapeDtypeStruct((B, K), jnp.int32),
        ],
        scratch_shapes=[
            pltpu.VMEM((B, NPAD), jnp.float32),
            pltpu.VMEM((B, D), jnp.float32),
        ],
    )(img, db)


NW = 32
BPW = (B * K) // NW
CH = 16
GM = 1920
GT = 128


def _sc_gather(table, tail, idx_flat):
    mesh = plsc.VectorSubcoreMesh(core_axis_name="c", subcore_axis_name="s")

    @functools.partial(
        pl.kernel,
        mesh=mesh,
        out_type=(
            jax.ShapeDtypeStruct((B * K, GM), jnp.float32),
            jax.ShapeDtypeStruct((B * K, GT), jnp.float32),
        ),
        scratch_types=[
            pltpu.VMEM((CH,), jnp.int32),
            pltpu.VMEM((CH, GM), jnp.float32),
            pltpu.VMEM((CH, GT), jnp.float32),
            pltpu.SemaphoreType.DMA,
            pltpu.SemaphoreType.DMA,
        ],
    )
    def k(table_hbm, tail_hbm, idx_hbm, out_hbm, tout_hbm, idx_v, rows_v,
          tail_v, sem, sem2):
        wid = lax.axis_index("s") * 2 + lax.axis_index("c")
        base = wid * BPW
        for c in range(BPW // CH):
            pltpu.sync_copy(idx_hbm.at[pl.ds(base + c * CH, CH)], idx_v)
            cp1 = pltpu.async_copy(table_hbm.at[idx_v, pl.ds(0, GM)], rows_v, sem)
            cp2 = pltpu.async_copy(tail_hbm.at[idx_v], tail_v, sem2)
            cp1.wait()
            cp2.wait()
            pltpu.sync_copy(rows_v, out_hbm.at[pl.ds(base + c * CH, CH)])
            pltpu.sync_copy(tail_v, tout_hbm.at[pl.ds(base + c * CH, CH)])

    return k(table, tail, idx_flat)


def _build_body(gath_ref, tail_ref, wflat_ref, wp, wp2, bp, img_ref, wi, bi,
                pos_ref, x0_ref):
    g = (jnp.dot(gath_ref[...], wp[0:GM, :], preferred_element_type=jnp.float32)
         + jnp.dot(tail_ref[...], wp2[...], preferred_element_type=jnp.float32))
    g = g * wflat_ref[...]
    imgf = jnp.dot(img_ref[...], wi[...], preferred_element_type=jnp.float32) + bi[...]
    x0_ref[0:B, :] = imgf + pos_ref[0:1, :]
    for s in range(1, S):
        x0_ref[s * B:(s + 1) * B, :] = (g[(s - 1) * B:s * B, :] + bp[...]
                                        + pos_ref[s:s + 1, :])


def _build_tokens(gath, tail, w_flat, img, p, pos):
    r2 = lambda a: a.reshape(1, -1)
    wp2 = jnp.pad(p['scrna_proj_w'][GM:G, :], ((0, GT - (G - GM)), (0, 0)))
    return _pc(
        _build_body,
        out_shape=jax.ShapeDtypeStruct((S * B, H), jnp.float32),
    )(gath, tail, w_flat, p['scrna_proj_w'], wp2, r2(p['scrna_proj_b']),
      img, p['img_proj_w'], r2(p['img_proj_b']), pos)


def _ln(x, g, b):
    mu = jnp.mean(x, axis=1, keepdims=True)
    var = jnp.mean((x - mu) ** 2, axis=1, keepdims=True)
    return g[...] * (x - mu) / jnp.sqrt(var + 1e-5) + b[...]


def _attn_body(x_ref, wqkv, bqkv, wo, bo, g1, b1, out_ref):
    x = x_ref[...]
    qkv = jnp.dot(x, wqkv[...], preferred_element_type=jnp.float32) + bqkv[...]
    q = qkv[:, 0:H]
    kk = qkv[:, H:2 * H]
    v = qkv[:, 2 * H:3 * H]

    r = lax.broadcasted_iota(jnp.int32, (H, NH), 0)
    c = lax.broadcasted_iota(jnp.int32, (H, NH), 1)
    M = (r // DH == c).astype(jnp.float32)
    rt = lax.broadcasted_iota(jnp.int32, (NH, H), 0)
    ct = lax.broadcasted_iota(jnp.int32, (NH, H), 1)
    MT = (rt == ct // DH).astype(jnp.float32)

    scale = 1.0 / 8.0
    o_parts = []
    for i in range(S):
        qi = q[i * B:(i + 1) * B, :]
        sij = []
        for j in range(S):
            kj = kk[j * B:(j + 1) * B, :]
            sij.append(jnp.dot(qi * kj, M, preferred_element_type=jnp.float32) * scale)
        m = sij[0]
        for j in range(1, S):
            m = jnp.maximum(m, sij[j])
        es = [jnp.exp(sv - m) for sv in sij]
        z = es[0]
        for j in range(1, S):
            z = z + es[j]
        zi = 1.0 / z
        oi = None
        for j in range(S):
            ab = jnp.dot(es[j] * zi, MT, preferred_element_type=jnp.float32)
            t = ab * v[j * B:(j + 1) * B, :]
            oi = t if oi is None else oi + t
        o_parts.append(oi)
    o = jnp.concatenate(o_parts, axis=0)
    attn = jnp.dot(o, wo[...], preferred_element_type=jnp.float32) + bo[...]
    out_ref[...] = _ln(x + attn, g1, b1)


def _ffn_body(x_ref, w1, b1, w2, b2, g2, bb2, out_ref):
    x = x_ref[...]
    h = jnp.maximum(jnp.dot(x, w1[...], preferred_element_type=jnp.float32) + b1[...], 0.0)
    f = jnp.dot(h, w2[...], preferred_element_type=jnp.float32) + b2[...]
    out_ref[...] = _ln(x + f, g2, bb2)


def _encoder_layer(x, lp):
    r2 = lambda a: a.reshape(1, -1)
    x = _pc(
        _attn_body,
        out_shape=jax.ShapeDtypeStruct((S * B, H), jnp.float32),
    )(x, lp['wqkv'], r2(lp['bqkv']), lp['wo'], r2(lp['bo']),
      r2(lp['ln1_g']), r2(lp['ln1_b']))
    x = _pc(
        _ffn_body,
        out_shape=jax.ShapeDtypeStruct((S * B, H), jnp.float32),
    )(x, lp['ffn_w1'], r2(lp['ffn_b1']), lp['ffn_w2'], r2(lp['ffn_b2']),
      r2(lp['ln2_g']), r2(lp['ln2_b']))
    return x


def _head_body(x_ref, w1, b1, w2, b2, out_ref):
    o = jnp.maximum(jnp.dot(x_ref[...], w1[...], preferred_element_type=jnp.float32) + b1[...], 0.0)
    out_ref[...] = jnp.dot(o, w2[...], preferred_element_type=jnp.float32) + b2[...]


def _head(x, p):
    r2 = lambda a: a.reshape(1, -1)
    return _pc(
        _head_body,
        out_shape=jax.ShapeDtypeStruct((B, G), jnp.float32),
    )(x, p['out_w1'], r2(p['out_b1']), p['out_w2'], r2(p['out_b2']))


def kernel(image_embeddings, scrna_embeddings, scrna_expressions, params):
    p = params
    logits = _classifier(image_embeddings, p)

    weights, top_idx = _retrieval(image_embeddings, scrna_embeddings)
    idx_flat = top_idx.T.reshape(B * K)
    w_flat = weights.T.reshape(B * K, 1)

    expr_tail = jnp.pad(scrna_expressions[:, GM:G], ((0, 0), (0, GT - (G - GM))))
    gath, gtail = _sc_gather(scrna_expressions, expr_tail, idx_flat)

    pos = p['pos_emb'][0, :S, :]
    x = _build_tokens(gath, gtail, w_flat, image_embeddings, p, pos)
    for lp in p['layers']:
        x = _encoder_layer(x, lp)
    gene = _head(x[0:B], p)
    return gene, logits


# speedup vs baseline: 2.3061x; 1.0022x over previous
import functools

import jax
import jax.numpy as jnp
from jax import lax
from jax.experimental import pallas as pl
from jax.experimental.pallas import tpu as pltpu
from jax.experimental.pallas import tpu_sc as plsc

B = 256
D = 768
N = 20000
G = 2000
H = 512
K = 10
S = 11
NH = 8
DH = H // NH

TILE = 2048
NPAD = 20480
NT = NPAD // TILE


def _pc(body, **kw):
    return pl.pallas_call(body, **kw)


def _cls_body(x_ref, w1, b1, g1, be1, w2, b2, g2, be2, w3, b3, out_ref):
    def bn(h, g, b):
        mu = jnp.mean(h, axis=0, keepdims=True)
        var = jnp.mean((h - mu) ** 2, axis=0, keepdims=True)
        return g[...] * (h - mu) / jnp.sqrt(var + 1e-5) + b[...]

    h = jnp.maximum(jnp.dot(x_ref[...], w1[...], preferred_element_type=jnp.float32) + b1[...], 0.0)
    h = bn(h, g1, be1)
    h = jnp.maximum(jnp.dot(h, w2[...], preferred_element_type=jnp.float32) + b2[...], 0.0)
    h = bn(h, g2, be2)
    out_ref[...] = jnp.dot(h, w3[...], preferred_element_type=jnp.float32) + b3[...]


def _classifier(x, p):
    r2 = lambda a: a.reshape(1, -1)
    return _pc(
        _cls_body,
        out_shape=jax.ShapeDtypeStruct((B, 100), jnp.float32),
    )(x, p['cls_w1'], r2(p['cls_b1']), r2(p['cls_g1']), r2(p['cls_be1']),
      p['cls_w2'], r2(p['cls_b2']), r2(p['cls_g2']), r2(p['cls_be2']),
      p['cls_w3'], r2(p['cls_b3']))


def _retr_body(img_ref, db_ref, w_ref, idx_ref, sims_ref, qn_ref):
    i = pl.program_id(0)

    @pl.when(i == 0)
    def _():
        x = img_ref[...]
        nrm = jnp.sqrt(jnp.sum(x * x, axis=1, keepdims=True))
        qn_ref[...] = x / (nrm + 1e-8)

    d = db_ref[...]
    nrm = jnp.sqrt(jnp.sum(d * d, axis=1, keepdims=True))
    dn = d / (nrm + 1e-8)
    blk = lax.dot_general(qn_ref[...], dn, (((1,), (1,)), ((), ())),
                          preferred_element_type=jnp.float32)
    sims_ref[:, pl.ds(i * TILE, TILE)] = blk

    @pl.when(i == NT - 1)
    def _():
        RB = 32
        for rb in range(B // RB):
            s = sims_ref[rb * RB:(rb + 1) * RB, :]
            colid = lax.broadcasted_iota(jnp.int32, (RB, NPAD), 1)
            s = jnp.where(colid < N, s, -jnp.inf)
            vals, idxs = [], []
            for _k in range(K):
                m = jnp.max(s, axis=1, keepdims=True)
                ix = jnp.min(jnp.where(s == m, colid, jnp.int32(2 ** 30)),
                             axis=1, keepdims=True)
                vals.append(m)
                idxs.append(ix)
                s = jnp.where(colid == ix, -jnp.inf, s)
            v = jnp.concatenate(vals, axis=1)
            mm = jnp.max(v, axis=1, keepdims=True)
            e = jnp.exp(v - mm)
            w_ref[rb * RB:(rb + 1) * RB, :] = e / jnp.sum(e, axis=1, keepdims=True)
            idx_ref[rb * RB:(rb + 1) * RB, :] = jnp.concatenate(idxs, axis=1)


def _retrieval(img, db):
    return _pc(
        _retr_body,
        grid=(NT,),
        in_specs=[
            pl.BlockSpec((B, D), lambda i: (0, 0)),
            pl.BlockSpec((TILE, D), lambda i: (i, 0)),
        ],
        out_specs=[
            pl.BlockSpec((B, K), lambda i: (0, 0)),
            pl.BlockSpec((B, K), lambda i: (0, 0)),
        ],
        out_shape=[
            jax.ShapeDtypeStruct((B, K), jnp.float32),
            jax.ShapeDtypeStruct((B, K), jnp.int32),
        ],
        scratch_shapes=[
            pltpu.VMEM((B, NPAD), jnp.float32),
            pltpu.VMEM((B, D), jnp.float32),
        ],
    )(img, db)


NW = 32
BPW = (B * K) // NW
CH = 16
GM = 1920
GT = 128


def _sc_gather(table, tail, idx_flat):
    mesh = plsc.VectorSubcoreMesh(core_axis_name="c", subcore_axis_name="s")

    @functools.partial(
        pl.kernel,
        mesh=mesh,
        out_type=(
            jax.ShapeDtypeStruct((B * K, GM), jnp.float32),
            jax.ShapeDtypeStruct((B * K, GT), jnp.float32),
        ),
        scratch_types=[
            pltpu.VMEM((CH,), jnp.int32),
            pltpu.VMEM((CH, GM), jnp.float32),
            pltpu.VMEM((CH, GT), jnp.float32),
            pltpu.SemaphoreType.DMA,
            pltpu.SemaphoreType.DMA,
        ],
    )
    def k(table_hbm, tail_hbm, idx_hbm, out_hbm, tout_hbm, idx_v, rows_v,
          tail_v, sem, sem2):
        wid = lax.axis_index("s") * 2 + lax.axis_index("c")
        base = wid * BPW
        for c in range(BPW // CH):
            pltpu.sync_copy(idx_hbm.at[pl.ds(base + c * CH, CH)], idx_v)
            cp1 = pltpu.async_copy(table_hbm.at[idx_v, pl.ds(0, GM)], rows_v, sem)
            cp2 = pltpu.async_copy(tail_hbm.at[idx_v], tail_v, sem2)
            cp1.wait()
            cp2.wait()
            pltpu.sync_copy(rows_v, out_hbm.at[pl.ds(base + c * CH, CH)])
            pltpu.sync_copy(tail_v, tout_hbm.at[pl.ds(base + c * CH, CH)])

    return k(table, tail, idx_flat)


def _bdot(a, b):
    return jnp.dot(a.astype(jnp.bfloat16), b.astype(jnp.bfloat16),
                   preferred_element_type=jnp.float32)


def _build_body(gath_ref, tail_ref, wflat_ref, wp, wp2, bp, img_ref, wi, bi,
                pos_ref, x0_ref):
    g = _bdot(gath_ref[...], wp[0:GM, :]) + _bdot(tail_ref[...], wp2[...])
    g = g * wflat_ref[...]
    imgf = _bdot(img_ref[...], wi[...]) + bi[...]
    x0_ref[0:B, :] = imgf + pos_ref[0:1, :]
    for s in range(1, S):
        x0_ref[s * B:(s + 1) * B, :] = (g[(s - 1) * B:s * B, :] + bp[...]
                                        + pos_ref[s:s + 1, :])


def _build_tokens(gath, tail, w_flat, img, p, pos):
    r2 = lambda a: a.reshape(1, -1)
    wp2 = jnp.pad(p['scrna_proj_w'][GM:G, :], ((0, GT - (G - GM)), (0, 0)))
    return _pc(
        _build_body,
        out_shape=jax.ShapeDtypeStruct((S * B, H), jnp.float32),
    )(gath, tail, w_flat, p['scrna_proj_w'], wp2, r2(p['scrna_proj_b']),
      img, p['img_proj_w'], r2(p['img_proj_b']), pos)


def _ln(x, g, b):
    mu = jnp.mean(x, axis=1, keepdims=True)
    var = jnp.mean((x - mu) ** 2, axis=1, keepdims=True)
    return g[...] * (x - mu) / jnp.sqrt(var + 1e-5) + b[...]


def _attn_body(x_ref, wqkv, bqkv, wo, bo, g1, b1, out_ref):
    x = x_ref[...]
    qkv = _bdot(x, wqkv[...]) + bqkv[...]
    q = qkv[:, 0:H].astype(jnp.bfloat16)
    kk = qkv[:, H:2 * H].astype(jnp.bfloat16)
    v = qkv[:, 2 * H:3 * H]

    r = lax.broadcasted_iota(jnp.int32, (H, NH), 0)
    c = lax.broadcasted_iota(jnp.int32, (H, NH), 1)
    M = (r // DH == c).astype(jnp.bfloat16)
    rt = lax.broadcasted_iota(jnp.int32, (NH, H), 0)
    ct = lax.broadcasted_iota(jnp.int32, (NH, H), 1)
    MT = (rt == ct // DH).astype(jnp.bfloat16)

    scale = 1.0 / 8.0
    o_parts = []
    for i in range(S):
        qi = q[i * B:(i + 1) * B, :]
        sij = []
        for j in range(S):
            kj = kk[j * B:(j + 1) * B, :]
            sij.append(jnp.dot(qi * kj, M, preferred_element_type=jnp.float32) * scale)
        m = sij[0]
        for j in range(1, S):
            m = jnp.maximum(m, sij[j])
        es = [jnp.exp(sv - m) for sv in sij]
        z = es[0]
        for j in range(1, S):
            z = z + es[j]
        zi = 1.0 / z
        oi = None
        for j in range(S):
            ab = jnp.dot((es[j] * zi).astype(jnp.bfloat16), MT,
                         preferred_element_type=jnp.float32)
            t = ab * v[j * B:(j + 1) * B, :]
            oi = t if oi is None else oi + t
        o_parts.append(oi)
    o = jnp.concatenate(o_parts, axis=0)
    attn = _bdot(o, wo[...]) + bo[...]
    out_ref[...] = _ln(x + attn, g1, b1)


def _ffn_body(x_ref, w1, b1, w2, b2, g2, bb2, out_ref):
    x = x_ref[...]
    h = jnp.maximum(_bdot(x, w1[...]) + b1[...], 0.0)
    f = _bdot(h, w2[...]) + b2[...]
    out_ref[...] = _ln(x + f, g2, bb2)


def _encoder_layer(x, lp):
    r2 = lambda a: a.reshape(1, -1)
    x = _pc(
        _attn_body,
        out_shape=jax.ShapeDtypeStruct((S * B, H), jnp.float32),
    )(x, lp['wqkv'], r2(lp['bqkv']), lp['wo'], r2(lp['bo']),
      r2(lp['ln1_g']), r2(lp['ln1_b']))
    x = _pc(
        _ffn_body,
        out_shape=jax.ShapeDtypeStruct((S * B, H), jnp.float32),
    )(x, lp['ffn_w1'], r2(lp['ffn_b1']), lp['ffn_w2'], r2(lp['ffn_b2']),
      r2(lp['ln2_g']), r2(lp['ln2_b']))
    return x


def _head_body(x_ref, w1, b1, w2, b2, out_ref):
    o = jnp.maximum(_bdot(x_ref[...], w1[...]) + b1[...], 0.0)
    out_ref[...] = _bdot(o, w2[...]) + b2[...]


def _head(x, p):
    r2 = lambda a: a.reshape(1, -1)
    return _pc(
        _head_body,
        out_shape=jax.ShapeDtypeStruct((B, G), jnp.float32),
    )(x, p['out_w1'], r2(p['out_b1']), p['out_w2'], r2(p['out_b2']))


def kernel(image_embeddings, scrna_embeddings, scrna_expressions, params):
    p = params
    logits = _classifier(image_embeddings, p)

    weights, top_idx = _retrieval(image_embeddings, scrna_embeddings)
    idx_flat = top_idx.T.reshape(B * K)
    w_flat = weights.T.reshape(B * K, 1)

    expr_tail = jnp.pad(scrna_expressions[:, GM:G], ((0, 0), (0, GT - (G - GM))))
    gath, gtail = _sc_gather(scrna_expressions, expr_tail, idx_flat)

    pos = p['pos_emb'][0, :S, :]
    x = _build_tokens(gath, gtail, w_flat, image_embeddings, p, pos)
    for lp in p['layers']:
        x = _encoder_layer(x, lp)
    gene = _head(x[0:B], p)
    return gene, logits


# speedup vs baseline: 2.4425x; 1.0592x over previous
import functools

import jax
import jax.numpy as jnp
from jax import lax
from jax.experimental import pallas as pl
from jax.experimental.pallas import tpu as pltpu
from jax.experimental.pallas import tpu_sc as plsc

B = 256
D = 768
N = 20000
G = 2000
H = 512
K = 10
S = 11
NH = 8
DH = H // NH

TILE = 2048
NPAD = 20480
NT = NPAD // TILE


def _pc(body, **kw):
    return pl.pallas_call(body, **kw)


def _retr_body(img_ref, db_ref, w1, b1, g1, be1, w2, b2, g2, be2, w3, b3,
               w_ref, idx_ref, logits_ref, sims_ref, qn_ref):
    i = pl.program_id(0)

    @pl.when(i == 0)
    def _():
        x = img_ref[...]
        nrm = jnp.sqrt(jnp.sum(x * x, axis=1, keepdims=True))
        qn_ref[...] = x / (nrm + 1e-8)

        def bn(h, g, b):
            mu = jnp.mean(h, axis=0, keepdims=True)
            var = jnp.mean((h - mu) ** 2, axis=0, keepdims=True)
            return g[...] * (h - mu) / jnp.sqrt(var + 1e-5) + b[...]

        h = jnp.maximum(jnp.dot(x, w1[...], preferred_element_type=jnp.float32) + b1[...], 0.0)
        h = bn(h, g1, be1)
        h = jnp.maximum(jnp.dot(h, w2[...], preferred_element_type=jnp.float32) + b2[...], 0.0)
        h = bn(h, g2, be2)
        logits_ref[...] = jnp.dot(h, w3[...], preferred_element_type=jnp.float32) + b3[...]

    d = db_ref[...]
    nrm = jnp.sqrt(jnp.sum(d * d, axis=1, keepdims=True))
    dn = d / (nrm + 1e-8)
    blk = lax.dot_general(qn_ref[...], dn, (((1,), (1,)), ((), ())),
                          preferred_element_type=jnp.float32)
    sims_ref[:, pl.ds(i * TILE, TILE)] = blk

    @pl.when(i == NT - 1)
    def _():
        RB = 32
        for rb in range(B // RB):
            s = sims_ref[rb * RB:(rb + 1) * RB, :]
            colid = lax.broadcasted_iota(jnp.int32, (RB, NPAD), 1)
            s = jnp.where(colid < N, s, -jnp.inf)
            vals, idxs = [], []
            for _k in range(K):
                m = jnp.max(s, axis=1, keepdims=True)
                ix = jnp.min(jnp.where(s == m, colid, jnp.int32(2 ** 30)),
                             axis=1, keepdims=True)
                vals.append(m)
                idxs.append(ix)
                s = jnp.where(colid == ix, -jnp.inf, s)
            v = jnp.concatenate(vals, axis=1)
            mm = jnp.max(v, axis=1, keepdims=True)
            e = jnp.exp(v - mm)
            w_ref[rb * RB:(rb + 1) * RB, :] = e / jnp.sum(e, axis=1, keepdims=True)
            idx_ref[rb * RB:(rb + 1) * RB, :] = jnp.concatenate(idxs, axis=1)


def _retrieval(img, db, p):
    r2 = lambda a: a.reshape(1, -1)
    full = lambda shape: pl.BlockSpec(shape, lambda i: tuple(0 for _ in shape))
    return _pc(
        _retr_body,
        grid=(NT,),
        compiler_params=pltpu.CompilerParams(vmem_limit_bytes=63 * 1024 * 1024),
        in_specs=[
            full((B, D)),
            pl.BlockSpec((TILE, D), lambda i: (i, 0)),
            full((D, 512)), full((1, 512)), full((1, 512)), full((1, 512)),
            full((512, 256)), full((1, 256)), full((1, 256)), full((1, 256)),
            full((256, 100)), full((1, 100)),
        ],
        out_specs=[
            full((B, K)),
            full((B, K)),
            full((B, 100)),
        ],
        out_shape=[
            jax.ShapeDtypeStruct((B, K), jnp.float32),
            jax.ShapeDtypeStruct((B, K), jnp.int32),
            jax.ShapeDtypeStruct((B, 100), jnp.float32),
        ],
        scratch_shapes=[
            pltpu.VMEM((B, NPAD), jnp.float32),
            pltpu.VMEM((B, D), jnp.float32),
        ],
    )(img, db, p['cls_w1'], r2(p['cls_b1']), r2(p['cls_g1']), r2(p['cls_be1']),
      p['cls_w2'], r2(p['cls_b2']), r2(p['cls_g2']), r2(p['cls_be2']),
      p['cls_w3'], r2(p['cls_b3']))


NW = 32
BPW = (B * K) // NW
CH = 16
GM = 1920
GT = 128


def _sc_gather(table, tail, idx_flat):
    mesh = plsc.VectorSubcoreMesh(core_axis_name="c", subcore_axis_name="s")

    @functools.partial(
        pl.kernel,
        mesh=mesh,
        out_type=(
            jax.ShapeDtypeStruct((B * K, GM), jnp.float32),
            jax.ShapeDtypeStruct((B * K, GT), jnp.float32),
        ),
        scratch_types=[
            pltpu.VMEM((CH,), jnp.int32),
            pltpu.VMEM((CH, GM), jnp.float32),
            pltpu.VMEM((CH, GT), jnp.float32),
            pltpu.SemaphoreType.DMA,
            pltpu.SemaphoreType.DMA,
        ],
    )
    def k(table_hbm, tail_hbm, idx_hbm, out_hbm, tout_hbm, idx_v, rows_v,
          tail_v, sem, sem2):
        wid = lax.axis_index("s") * 2 + lax.axis_index("c")
        base = wid * BPW
        for c in range(BPW // CH):
            pltpu.sync_copy(idx_hbm.at[pl.ds(base + c * CH, CH)], idx_v)
            cp1 = pltpu.async_copy(table_hbm.at[idx_v, pl.ds(0, GM)], rows_v, sem)
            cp2 = pltpu.async_copy(tail_hbm.at[idx_v], tail_v, sem2)
            cp1.wait()
            cp2.wait()
            pltpu.sync_copy(rows_v, out_hbm.at[pl.ds(base + c * CH, CH)])
            pltpu.sync_copy(tail_v, tout_hbm.at[pl.ds(base + c * CH, CH)])

    return k(table, tail, idx_flat)


def _bdot(a, b):
    return jnp.dot(a.astype(jnp.bfloat16), b.astype(jnp.bfloat16),
                   preferred_element_type=jnp.float32)


def _build_body(gath_ref, tail_ref, wflat_ref, wp, wp2, bp, img_ref, wi, bi,
                pos_ref, x0_ref):
    g = _bdot(gath_ref[...], wp[0:GM, :]) + _bdot(tail_ref[...], wp2[...])
    g = g * wflat_ref[...]
    imgf = _bdot(img_ref[...], wi[...]) + bi[...]
    x0_ref[0:B, :] = imgf + pos_ref[0:1, :]
    for s in range(1, S):
        x0_ref[s * B:(s + 1) * B, :] = (g[(s - 1) * B:s * B, :] + bp[...]
                                        + pos_ref[s:s + 1, :])


def _build_tokens(gath, tail, w_flat, img, p, pos):
    r2 = lambda a: a.reshape(1, -1)
    wp2 = jnp.pad(p['scrna_proj_w'][GM:G, :], ((0, GT - (G - GM)), (0, 0)))
    return _pc(
        _build_body,
        out_shape=jax.ShapeDtypeStruct((S * B, H), jnp.float32),
    )(gath, tail, w_flat, p['scrna_proj_w'], wp2, r2(p['scrna_proj_b']),
      img, p['img_proj_w'], r2(p['img_proj_b']), pos)


def _ln(x, g, b):
    mu = jnp.mean(x, axis=1, keepdims=True)
    var = jnp.mean((x - mu) ** 2, axis=1, keepdims=True)
    return g[...] * (x - mu) / jnp.sqrt(var + 1e-5) + b[...]


def _attn_core(x, wqkv, bqkv, wo, bo, g1, b1):
    xb = x.astype(jnp.bfloat16)
    q = (jnp.dot(xb, wqkv[:, 0:H].astype(jnp.bfloat16),
                 preferred_element_type=jnp.float32)
         + bqkv[0:1, 0:H]).astype(jnp.bfloat16)
    kk = (jnp.dot(xb, wqkv[:, H:2 * H].astype(jnp.bfloat16),
                  preferred_element_type=jnp.float32)
          + bqkv[0:1, H:2 * H]).astype(jnp.bfloat16)
    v = jnp.dot(xb, wqkv[:, 2 * H:3 * H].astype(jnp.bfloat16),
                preferred_element_type=jnp.float32) + bqkv[0:1, 2 * H:3 * H]

    r = lax.broadcasted_iota(jnp.int32, (H, NH), 0)
    c = lax.broadcasted_iota(jnp.int32, (H, NH), 1)
    M = (r // DH == c).astype(jnp.bfloat16)
    rt = lax.broadcasted_iota(jnp.int32, (NH, H), 0)
    ct = lax.broadcasted_iota(jnp.int32, (NH, H), 1)
    MT = (rt == ct // DH).astype(jnp.bfloat16)

    scale = 1.0 / 8.0
    o_parts = []
    for i in range(S):
        qi = q[i * B:(i + 1) * B, :]
        sij = []
        for j in range(S):
            kj = kk[j * B:(j + 1) * B, :]
            sij.append(jnp.dot(qi * kj, M, preferred_element_type=jnp.float32) * scale)
        m = sij[0]
        for j in range(1, S):
            m = jnp.maximum(m, sij[j])
        es = [jnp.exp(sv - m) for sv in sij]
        z = es[0]
        for j in range(1, S):
            z = z + es[j]
        zi = 1.0 / z
        oi = None
        for j in range(S):
            ab = jnp.dot((es[j] * zi).astype(jnp.bfloat16), MT,
                         preferred_element_type=jnp.float32)
            t = ab * v[j * B:(j + 1) * B, :]
            oi = t if oi is None else oi + t
        o_parts.append(oi)
    o = jnp.concatenate(o_parts, axis=0)
    attn = _bdot(o, wo[...]) + bo[...]
    return _ln(x + attn, g1, b1)


def _ffn_core(x, w1, b1, w2, b2, g2, bb2):
    xb = x.astype(jnp.bfloat16)
    FH = 2 * H
    f = None
    for c in range(2):
        h = jnp.maximum(
            jnp.dot(xb, w1[:, c * FH:(c + 1) * FH].astype(jnp.bfloat16),
                    preferred_element_type=jnp.float32)
            + b1[0:1, c * FH:(c + 1) * FH], 0.0).astype(jnp.bfloat16)
        fc = jnp.dot(h, w2[c * FH:(c + 1) * FH, :].astype(jnp.bfloat16),
                     preferred_element_type=jnp.float32)
        f = fc if f is None else f + fc
    return _ln(x + f + b2[...], g2, bb2)


def _layer_body(x_ref, wqkv, bqkv, wo, bo, g1, b1, w1, bf1, w2, bf2, g2, bb2,
                out_ref):
    x1 = _attn_core(x_ref[...], wqkv, bqkv, wo, bo, g1, b1)
    out_ref[...] = _ffn_core(x1, w1, bf1, w2, bf2, g2, bb2)


def _layer_head_body(x_ref, wqkv, bqkv, wo, bo, g1, b1, w1, bf1, w2, bf2, g2,
                     bb2, hw1, hb1, hw2, hb2, out_ref, gene_ref):
    x1 = _attn_core(x_ref[...], wqkv, bqkv, wo, bo, g1, b1)
    x2 = _ffn_core(x1, w1, bf1, w2, bf2, g2, bb2)
    out_ref[...] = x2
    o = jnp.maximum(_bdot(x2[0:B, :], hw1[...]) + hb1[...], 0.0)
    gene_ref[...] = _bdot(o, hw2[...]) + hb2[...]


def _encoder_layer(x, lp):
    r2 = lambda a: a.reshape(1, -1)
    return _pc(
        _layer_body,
        out_shape=jax.ShapeDtypeStruct((S * B, H), jnp.float32),
        input_output_aliases={0: 0},
        compiler_params=pltpu.CompilerParams(vmem_limit_bytes=63 * 1024 * 1024),
    )(x, lp['wqkv'], r2(lp['bqkv']), lp['wo'], r2(lp['bo']),
      r2(lp['ln1_g']), r2(lp['ln1_b']),
      lp['ffn_w1'], r2(lp['ffn_b1']), lp['ffn_w2'], r2(lp['ffn_b2']),
      r2(lp['ln2_g']), r2(lp['ln2_b']))


def _encoder_layer_head(x, lp, p):
    r2 = lambda a: a.reshape(1, -1)
    _, gene = _pc(
        _layer_head_body,
        out_shape=[
            jax.ShapeDtypeStruct((S * B, H), jnp.float32),
            jax.ShapeDtypeStruct((B, G), jnp.float32),
        ],
        input_output_aliases={0: 0},
        compiler_params=pltpu.CompilerParams(vmem_limit_bytes=63 * 1024 * 1024),
    )(x, lp['wqkv'], r2(lp['bqkv']), lp['wo'], r2(lp['bo']),
      r2(lp['ln1_g']), r2(lp['ln1_b']),
      lp['ffn_w1'], r2(lp['ffn_b1']), lp['ffn_w2'], r2(lp['ffn_b2']),
      r2(lp['ln2_g']), r2(lp['ln2_b']),
      p['out_w1'], r2(p['out_b1']), p['out_w2'], r2(p['out_b2']))
    return gene


def kernel(image_embeddings, scrna_embeddings, scrna_expressions, params):
    p = params
    weights, top_idx, logits = _retrieval(image_embeddings, scrna_embeddings, p)
    idx_flat = top_idx.T.reshape(B * K)
    w_flat = weights.T.reshape(B * K, 1)

    expr_tail = jnp.pad(scrna_expressions[:, GM:G], ((0, 0), (0, GT - (G - GM))))
    gath, gtail = _sc_gather(scrna_expressions, expr_tail, idx_flat)

    pos = p['pos_emb'][0, :S, :]
    x = _build_tokens(gath, gtail, w_flat, image_embeddings, p, pos)
    for lp in p['layers'][:-1]:
        x = _encoder_layer(x, lp)
    gene = _encoder_layer_head(x, p['layers'][-1], p)
    return gene, logits


# speedup vs baseline: 2.9863x; 1.2226x over previous
import functools

import jax
import jax.numpy as jnp
from jax import lax
from jax.experimental import pallas as pl
from jax.experimental.pallas import tpu as pltpu
from jax.experimental.pallas import tpu_sc as plsc

B = 256
D = 768
N = 20000
G = 2000
H = 512
K = 10
S = 11
NH = 8
DH = H // NH

TILE = 2048
NPAD = 20480
NT = NPAD // TILE


def _pc(body, **kw):
    return pl.pallas_call(body, **kw)


def _retr_body(img_ref, db_ref, w1, b1, g1, be1, w2, b2, g2, be2, w3, b3,
               w_ref, idx_ref, logits_ref, sims_ref, qn_ref):
    i = pl.program_id(0)

    @pl.when(i == 0)
    def _():
        x = img_ref[...]
        nrm = jnp.sqrt(jnp.sum(x * x, axis=1, keepdims=True))
        qn_ref[...] = x / (nrm + 1e-8)

        def bn(h, g, b):
            mu = jnp.mean(h, axis=0, keepdims=True)
            var = jnp.mean((h - mu) ** 2, axis=0, keepdims=True)
            return g[...] * (h - mu) / jnp.sqrt(var + 1e-5) + b[...]

        h = jnp.maximum(jnp.dot(x, w1[...], preferred_element_type=jnp.float32) + b1[...], 0.0)
        h = bn(h, g1, be1)
        h = jnp.maximum(jnp.dot(h, w2[...], preferred_element_type=jnp.float32) + b2[...], 0.0)
        h = bn(h, g2, be2)
        logits_ref[...] = jnp.dot(h, w3[...], preferred_element_type=jnp.float32) + b3[...]

    d = db_ref[...]
    nrm = jnp.sqrt(jnp.sum(d * d, axis=1, keepdims=True))
    dn = d / (nrm + 1e-8)
    blk = lax.dot_general(qn_ref[...], dn, (((1,), (1,)), ((), ())),
                          preferred_element_type=jnp.float32)
    sims_ref[:, pl.ds(i * TILE, TILE)] = blk

    @pl.when(i == NT - 1)
    def _():
        RB = 32
        for rb in range(B // RB):
            s = sims_ref[rb * RB:(rb + 1) * RB, :]
            colid = lax.broadcasted_iota(jnp.int32, (RB, NPAD), 1)
            s = jnp.where(colid < N, s, -jnp.inf)
            vals, idxs = [], []
            for _k in range(K):
                m = jnp.max(s, axis=1, keepdims=True)
                ix = jnp.min(jnp.where(s == m, colid, jnp.int32(2 ** 30)),
                             axis=1, keepdims=True)
                vals.append(m)
                idxs.append(ix)
                s = jnp.where(colid == ix, -jnp.inf, s)
            v = jnp.concatenate(vals, axis=1)
            mm = jnp.max(v, axis=1, keepdims=True)
            e = jnp.exp(v - mm)
            w_ref[rb * RB:(rb + 1) * RB, :] = e / jnp.sum(e, axis=1, keepdims=True)
            idx_ref[rb * RB:(rb + 1) * RB, :] = jnp.concatenate(idxs, axis=1)


def _retrieval(img, db, p):
    r2 = lambda a: a.reshape(1, -1)
    full = lambda shape: pl.BlockSpec(shape, lambda i: tuple(0 for _ in shape))
    return _pc(
        _retr_body,
        grid=(NT,),
        compiler_params=pltpu.CompilerParams(vmem_limit_bytes=63 * 1024 * 1024),
        in_specs=[
            full((B, D)),
            pl.BlockSpec((TILE, D), lambda i: (i, 0)),
            full((D, 512)), full((1, 512)), full((1, 512)), full((1, 512)),
            full((512, 256)), full((1, 256)), full((1, 256)), full((1, 256)),
            full((256, 100)), full((1, 100)),
        ],
        out_specs=[
            full((B, K)),
            full((B, K)),
            full((B, 100)),
        ],
        out_shape=[
            jax.ShapeDtypeStruct((B, K), jnp.float32),
            jax.ShapeDtypeStruct((B, K), jnp.int32),
            jax.ShapeDtypeStruct((B, 100), jnp.float32),
        ],
        scratch_shapes=[
            pltpu.VMEM((B, NPAD), jnp.float32),
            pltpu.VMEM((B, D), jnp.float32),
        ],
    )(img, db, p['cls_w1'], r2(p['cls_b1']), r2(p['cls_g1']), r2(p['cls_be1']),
      p['cls_w2'], r2(p['cls_b2']), r2(p['cls_g2']), r2(p['cls_be2']),
      p['cls_w3'], r2(p['cls_b3']))


PTILE = 2048
PNT = 10


def _proj_body(et_ref, wp_ref, p_ref):
    eb = et_ref[...].astype(jnp.bfloat16)
    wb = wp_ref[...].astype(jnp.bfloat16)
    p_ref[...] = lax.dot_general(eb, wb, (((0,), (0,)), ((), ())),
                                 preferred_element_type=jnp.float32)


def _project(expr_t, wp):
    return _pc(
        _proj_body,
        grid=(PNT,),
        in_specs=[
            pl.BlockSpec((G, PTILE), lambda i: (0, i)),
            pl.BlockSpec((G, H), lambda i: (0, 0)),
        ],
        out_specs=pl.BlockSpec((PTILE, H), lambda i: (i, 0)),
        out_shape=jax.ShapeDtypeStruct((N, H), jnp.float32),
        compiler_params=pltpu.CompilerParams(vmem_limit_bytes=63 * 1024 * 1024),
    )(expr_t, wp)


NW = 32
BPW = (B * K) // NW
CH = 16


def _sc_gather(table, idx_flat):
    mesh = plsc.VectorSubcoreMesh(core_axis_name="c", subcore_axis_name="s")

    @functools.partial(
        pl.kernel,
        mesh=mesh,
        out_type=jax.ShapeDtypeStruct((B * K, H), jnp.float32),
        scratch_types=[
            pltpu.VMEM((CH,), jnp.int32),
            pltpu.VMEM((CH, H), jnp.float32),
            pltpu.SemaphoreType.DMA,
        ],
    )
    def k(table_hbm, idx_hbm, out_hbm, idx_v, rows_v, sem):
        wid = lax.axis_index("s") * 2 + lax.axis_index("c")
        base = wid * BPW
        for c in range(BPW // CH):
            pltpu.sync_copy(idx_hbm.at[pl.ds(base + c * CH, CH)], idx_v)
            pltpu.async_copy(table_hbm.at[idx_v], rows_v, sem).wait()
            pltpu.sync_copy(rows_v, out_hbm.at[pl.ds(base + c * CH, CH)])

    return k(table, idx_flat)


def _bdot(a, b):
    return jnp.dot(a.astype(jnp.bfloat16), b.astype(jnp.bfloat16),
                   preferred_element_type=jnp.float32)


def _build_body(gath_ref, wflat_ref, bp, img_ref, wi, bi, pos_ref, x0_ref):
    g = gath_ref[...] * wflat_ref[...]
    imgf = _bdot(img_ref[...], wi[...]) + bi[...]
    x0_ref[0:B, :] = imgf + pos_ref[0:1, :]
    for s in range(1, S):
        x0_ref[s * B:(s + 1) * B, :] = (g[(s - 1) * B:s * B, :] + bp[...]
                                        + pos_ref[s:s + 1, :])


def _build_tokens(gath, w_flat, img, p, pos):
    r2 = lambda a: a.reshape(1, -1)
    return _pc(
        _build_body,
        out_shape=jax.ShapeDtypeStruct((S * B, H), jnp.float32),
    )(gath, w_flat, r2(p['scrna_proj_b']),
      img, p['img_proj_w'], r2(p['img_proj_b']), pos)


def _ln(x, g, b):
    mu = jnp.mean(x, axis=1, keepdims=True)
    var = jnp.mean((x - mu) ** 2, axis=1, keepdims=True)
    return g[...] * (x - mu) / jnp.sqrt(var + 1e-5) + b[...]


def _attn_core(x, wqkv, bqkv, wo, bo, g1, b1):
    xb = x.astype(jnp.bfloat16)
    q = (jnp.dot(xb, wqkv[:, 0:H].astype(jnp.bfloat16),
                 preferred_element_type=jnp.float32)
         + bqkv[0:1, 0:H]).astype(jnp.bfloat16)
    kk = (jnp.dot(xb, wqkv[:, H:2 * H].astype(jnp.bfloat16),
                  preferred_element_type=jnp.float32)
          + bqkv[0:1, H:2 * H]).astype(jnp.bfloat16)
    v = jnp.dot(xb, wqkv[:, 2 * H:3 * H].astype(jnp.bfloat16),
                preferred_element_type=jnp.float32) + bqkv[0:1, 2 * H:3 * H]

    r = lax.broadcasted_iota(jnp.int32, (H, NH), 0)
    c = lax.broadcasted_iota(jnp.int32, (H, NH), 1)
    M = (r // DH == c).astype(jnp.bfloat16)
    rt = lax.broadcasted_iota(jnp.int32, (NH, H), 0)
    ct = lax.broadcasted_iota(jnp.int32, (NH, H), 1)
    MT = (rt == ct // DH).astype(jnp.bfloat16)

    scale = 1.0 / 8.0
    o_parts = []
    for i in range(S):
        qi = q[i * B:(i + 1) * B, :]
        sij = []
        for j in range(S):
            kj = kk[j * B:(j + 1) * B, :]
            sij.append(jnp.dot(qi * kj, M, preferred_element_type=jnp.float32) * scale)
        m = sij[0]
        for j in range(1, S):
            m = jnp.maximum(m, sij[j])
        es = [jnp.exp(sv - m) for sv in sij]
        z = es[0]
        for j in range(1, S):
            z = z + es[j]
        zi = 1.0 / z
        oi = None
        for j in range(S):
            ab = jnp.dot((es[j] * zi).astype(jnp.bfloat16), MT,
                         preferred_element_type=jnp.float32)
            t = ab * v[j * B:(j + 1) * B, :]
            oi = t if oi is None else oi + t
        o_parts.append(oi)
    o = jnp.concatenate(o_parts, axis=0)
    attn = _bdot(o, wo[...]) + bo[...]
    return _ln(x + attn, g1, b1)


def _ffn_core(x, w1, b1, w2, b2, g2, bb2):
    xb = x.astype(jnp.bfloat16)
    FH = 2 * H
    f = None
    for c in range(2):
        h = jnp.maximum(
            jnp.dot(xb, w1[:, c * FH:(c + 1) * FH].astype(jnp.bfloat16),
                    preferred_element_type=jnp.float32)
            + b1[0:1, c * FH:(c + 1) * FH], 0.0).astype(jnp.bfloat16)
        fc = jnp.dot(h, w2[c * FH:(c + 1) * FH, :].astype(jnp.bfloat16),
                     preferred_element_type=jnp.float32)
        f = fc if f is None else f + fc
    return _ln(x + f + b2[...], g2, bb2)


def _layer_body(x_ref, wqkv, bqkv, wo, bo, g1, b1, w1, bf1, w2, bf2, g2, bb2,
                out_ref):
    x1 = _attn_core(x_ref[...], wqkv, bqkv, wo, bo, g1, b1)
    out_ref[...] = _ffn_core(x1, w1, bf1, w2, bf2, g2, bb2)


def _layer_head_body(x_ref, wqkv, bqkv, wo, bo, g1, b1, w1, bf1, w2, bf2, g2,
                     bb2, hw1, hb1, hw2, hb2, out_ref, gene_ref):
    x1 = _attn_core(x_ref[...], wqkv, bqkv, wo, bo, g1, b1)
    x2 = _ffn_core(x1, w1, bf1, w2, bf2, g2, bb2)
    out_ref[...] = x2
    o = jnp.maximum(_bdot(x2[0:B, :], hw1[...]) + hb1[...], 0.0)
    gene_ref[...] = _bdot(o, hw2[...]) + hb2[...]


def _encoder_layer(x, lp):
    r2 = lambda a: a.reshape(1, -1)
    return _pc(
        _layer_body,
        out_shape=jax.ShapeDtypeStruct((S * B, H), jnp.float32),
        input_output_aliases={0: 0},
        compiler_params=pltpu.CompilerParams(vmem_limit_bytes=63 * 1024 * 1024),
    )(x, lp['wqkv'], r2(lp['bqkv']), lp['wo'], r2(lp['bo']),
      r2(lp['ln1_g']), r2(lp['ln1_b']),
      lp['ffn_w1'], r2(lp['ffn_b1']), lp['ffn_w2'], r2(lp['ffn_b2']),
      r2(lp['ln2_g']), r2(lp['ln2_b']))


def _encoder_layer_head(x, lp, p):
    r2 = lambda a: a.reshape(1, -1)
    _, gene = _pc(
        _layer_head_body,
        out_shape=[
            jax.ShapeDtypeStruct((S * B, H), jnp.float32),
            jax.ShapeDtypeStruct((B, G), jnp.float32),
        ],
        input_output_aliases={0: 0},
        compiler_params=pltpu.CompilerParams(vmem_limit_bytes=63 * 1024 * 1024),
    )(x, lp['wqkv'], r2(lp['bqkv']), lp['wo'], r2(lp['bo']),
      r2(lp['ln1_g']), r2(lp['ln1_b']),
      lp['ffn_w1'], r2(lp['ffn_b1']), lp['ffn_w2'], r2(lp['ffn_b2']),
      r2(lp['ln2_g']), r2(lp['ln2_b']),
      p['out_w1'], r2(p['out_b1']), p['out_w2'], r2(p['out_b2']))
    return gene


def kernel(image_embeddings, scrna_embeddings, scrna_expressions, params):
    p = params
    weights, top_idx, logits = _retrieval(image_embeddings, scrna_embeddings, p)
    idx_flat = top_idx.T.reshape(B * K)
    w_flat = weights.T.reshape(B * K, 1)

    proj = _project(scrna_expressions.T, p['scrna_proj_w'])
    gath = _sc_gather(proj, idx_flat)

    pos = p['pos_emb'][0, :S, :]
    x = _build_tokens(gath, w_flat, image_embeddings, p, pos)
    for lp in p['layers'][:-1]:
        x = _encoder_layer(x, lp)
    gene = _encoder_layer_head(x, p['layers'][-1], p)
    return gene, logits
